# baseline (speedup 1.0000x reference)
import jax
import jax.numpy as jnp
from jax.experimental import pallas as pl
from jax.experimental.pallas import tpu as pltpu


def _mm_pipe(x_hbm, w_hbm, b_hbm, o_hbm, xbuf, wbuf, obuf, bbuf, sx, sw, so, sb):
    T = x_hbm.shape[0]

    def in_copies(t, slot):
        return (
            pltpu.make_async_copy(x_hbm.at[t], xbuf.at[slot], sx.at[slot]),
            pltpu.make_async_copy(w_hbm.at[t], wbuf.at[slot], sw.at[slot]),
        )

    pltpu.make_async_copy(b_hbm, bbuf, sb).start()
    for c in in_copies(0, 0):
        c.start()
    for c in in_copies(1, 1):
        c.start()
    pltpu.make_async_copy(b_hbm, bbuf, sb).wait()
    for t in range(T):
        slot = t % 2
        for c in in_copies(t, slot):
            c.wait()
        if t >= 2:
            pltpu.make_async_copy(obuf.at[slot], o_hbm.at[t - 2], so.at[slot]).wait()
        obuf[slot] = (
            jnp.dot(xbuf[slot], wbuf[slot], preferred_element_type=jnp.float32)
            + bbuf[t]
        )
        pltpu.make_async_copy(obuf.at[slot], o_hbm.at[t], so.at[slot]).start()
        if t + 2 < T:
            for c in in_copies(t + 2, slot):
                c.start()
    for t in (T - 2, T - 1):
        slot = t % 2
        pltpu.make_async_copy(obuf.at[slot], o_hbm.at[t], so.at[slot]).wait()


def kernel(inputs, w, b):
    T, B, D_IN = inputs.shape
    D_OUT = w.shape[2]
    b3 = b.reshape(T, 1, D_OUT)
    return pl.pallas_call(
        _mm_pipe,
        in_specs=[
            pl.BlockSpec(memory_space=pltpu.HBM),
            pl.BlockSpec(memory_space=pltpu.HBM),
            pl.BlockSpec(memory_space=pltpu.HBM),
        ],
        out_specs=pl.BlockSpec(memory_space=pltpu.HBM),
        out_shape=jax.ShapeDtypeStruct((T, B, D_OUT), jnp.float32),
        scratch_shapes=[
            pltpu.VMEM((2, B, D_IN), jnp.float32),
            pltpu.VMEM((2, D_IN, D_OUT), jnp.float32),
            pltpu.VMEM((2, B, D_OUT), jnp.float32),
            pltpu.VMEM((T, 1, D_OUT), jnp.float32),
            pltpu.SemaphoreType.DMA((2,)),
            pltpu.SemaphoreType.DMA((2,)),
            pltpu.SemaphoreType.DMA((2,)),
            pltpu.SemaphoreType.DMA,
        ],
    )(inputs, w, b3)


# speedup vs baseline: 1.0134x; 1.0134x over previous
import jax
import jax.numpy as jnp
from jax.experimental import pallas as pl
from jax.experimental.pallas import tpu as pltpu


def _mm_pipe(x_hbm, w_hbm, b_hbm, o_hbm, xbuf, wbuf, obuf, bbuf, sx, sw, so, sb):
    T = x_hbm.shape[0]

    def in_copies(t, slot):
        return (
            pltpu.make_async_copy(x_hbm.at[t], xbuf.at[slot], sx.at[slot]),
            pltpu.make_async_copy(w_hbm.at[t], wbuf.at[slot], sw.at[slot]),
        )

    pltpu.make_async_copy(b_hbm, bbuf, sb).start()
    for t0 in range(3):
        for c in in_copies(t0, t0):
            c.start()
    pltpu.make_async_copy(b_hbm, bbuf, sb).wait()
    for t in range(T):
        slot = t % 3
        oslot = t % 2
        for c in in_copies(t, slot):
            c.wait()
        if t >= 2:
            pltpu.make_async_copy(obuf.at[oslot], o_hbm.at[t - 2], so.at[oslot]).wait()
        obuf[oslot] = (
            jnp.dot(xbuf[slot], wbuf[slot], preferred_element_type=jnp.float32)
            + bbuf[t]
        )
        pltpu.make_async_copy(obuf.at[oslot], o_hbm.at[t], so.at[oslot]).start()
        if t + 3 < T:
            for c in in_copies(t + 3, slot):
                c.start()
    for t in (T - 2, T - 1):
        oslot = t % 2
        pltpu.make_async_copy(obuf.at[oslot], o_hbm.at[t], so.at[oslot]).wait()


def kernel(inputs, w, b):
    T, B, D_IN = inputs.shape
    D_OUT = w.shape[2]
    b3 = b.reshape(T, 1, D_OUT)
    return pl.pallas_call(
        _mm_pipe,
        in_specs=[
            pl.BlockSpec(memory_space=pltpu.HBM),
            pl.BlockSpec(memory_space=pltpu.HBM),
            pl.BlockSpec(memory_space=pltpu.HBM),
        ],
        out_specs=pl.BlockSpec(memory_space=pltpu.HBM),
        out_shape=jax.ShapeDtypeStruct((T, B, D_OUT), jnp.float32),
        scratch_shapes=[
            pltpu.VMEM((3, B, D_IN), jnp.float32),
            pltpu.VMEM((3, D_IN, D_OUT), jnp.float32),
            pltpu.VMEM((2, B, D_OUT), jnp.float32),
            pltpu.VMEM((T, 1, D_OUT), jnp.float32),
            pltpu.SemaphoreType.DMA((3,)),
            pltpu.SemaphoreType.DMA((3,)),
            pltpu.SemaphoreType.DMA((2,)),
            pltpu.SemaphoreType.DMA,
        ],
    )(inputs, w, b3)


# speedup vs baseline: 1.0257x; 1.0121x over previous
import jax
import jax.numpy as jnp
from jax.experimental import pallas as pl
from jax.experimental.pallas import tpu as pltpu


def _mm_pipe(x_hbm, w_hbm, b_hbm, o_hbm, xbuf, wbuf, obuf, bbuf, sx, sw, so, sb):
    T = x_hbm.shape[0]

    def in_copies(t, slot):
        return (
            pltpu.make_async_copy(x_hbm.at[t], xbuf.at[slot], sx.at[slot]),
            pltpu.make_async_copy(w_hbm.at[t], wbuf.at[slot], sw.at[slot]),
        )

    pltpu.make_async_copy(b_hbm, bbuf, sb).start()
    for t0 in range(3):
        for c in in_copies(t0, t0):
            c.start()
    pltpu.make_async_copy(b_hbm, bbuf, sb).wait()
    for t in range(T):
        slot = t % 3
        oslot = t % 2
        for c in in_copies(t, slot):
            c.wait()
        if t >= 2:
            pltpu.make_async_copy(obuf.at[oslot], o_hbm.at[t - 2], so.at[oslot]).wait()
        if t < T - 1:
            obuf[oslot] = (
                jnp.dot(xbuf[slot], wbuf[slot], preferred_element_type=jnp.float32)
                + bbuf[t]
            )
            pltpu.make_async_copy(obuf.at[oslot], o_hbm.at[t], so.at[oslot]).start()
        else:
            H = x_hbm.shape[1] // 2
            for h in range(2):
                rows = pl.ds(h * H, H)
                obuf[oslot, rows] = (
                    jnp.dot(xbuf[slot, rows], wbuf[slot],
                            preferred_element_type=jnp.float32)
                    + bbuf[t]
                )
                pltpu.make_async_copy(
                    obuf.at[oslot, rows], o_hbm.at[t, rows], so.at[oslot]
                ).start()
        if t + 3 < T:
            for c in in_copies(t + 3, slot):
                c.start()
    pltpu.make_async_copy(
        obuf.at[(T - 2) % 2], o_hbm.at[T - 2], so.at[(T - 2) % 2]
    ).wait()
    H = x_hbm.shape[1] // 2
    for h in range(2):
        rows = pl.ds(h * H, H)
        pltpu.make_async_copy(
            obuf.at[(T - 1) % 2, rows], o_hbm.at[T - 1, rows], so.at[(T - 1) % 2]
        ).wait()


def kernel(inputs, w, b):
    T, B, D_IN = inputs.shape
    D_OUT = w.shape[2]
    b3 = b.reshape(T, 1, D_OUT)
    return pl.pallas_call(
        _mm_pipe,
        in_specs=[
            pl.BlockSpec(memory_space=pltpu.HBM),
            pl.BlockSpec(memory_space=pltpu.HBM),
            pl.BlockSpec(memory_space=pltpu.HBM),
        ],
        out_specs=pl.BlockSpec(memory_space=pltpu.HBM),
        out_shape=jax.ShapeDtypeStruct((T, B, D_OUT), jnp.float32),
        scratch_shapes=[
            pltpu.VMEM((3, B, D_IN), jnp.float32),
            pltpu.VMEM((3, D_IN, D_OUT), jnp.float32),
            pltpu.VMEM((2, B, D_OUT), jnp.float32),
            pltpu.VMEM((T, 1, D_OUT), jnp.float32),
            pltpu.SemaphoreType.DMA((3,)),
            pltpu.SemaphoreType.DMA((3,)),
            pltpu.SemaphoreType.DMA((2,)),
            pltpu.SemaphoreType.DMA,
        ],
    )(inputs, w, b3)


# speedup vs baseline: 1.0270x; 1.0013x over previous
import jax
import jax.numpy as jnp
from jax.experimental import pallas as pl
from jax.experimental.pallas import tpu as pltpu


def _mm_pipe(x_hbm, w_hbm, b_hbm, o_hbm, xbuf, wbuf, obuf, bbuf, sx, sw, so, sb):
    T = x_hbm.shape[0]

    def in_copies(t, slot):
        return (
            pltpu.make_async_copy(x_hbm.at[t], xbuf.at[slot], sx.at[slot]),
            pltpu.make_async_copy(w_hbm.at[t], wbuf.at[slot], sw.at[slot]),
        )

    pltpu.make_async_copy(b_hbm, bbuf, sb).start()
    for t0 in range(3):
        for c in in_copies(t0, t0):
            c.start()
    pltpu.make_async_copy(b_hbm, bbuf, sb).wait()
    for t in range(T):
        slot = t % 3
        oslot = t % 3
        for c in in_copies(t, slot):
            c.wait()
        if t >= 3:
            pltpu.make_async_copy(obuf.at[oslot], o_hbm.at[t - 3], so.at[oslot]).wait()
        if t < T - 1:
            obuf[oslot] = (
                jnp.dot(xbuf[slot], wbuf[slot], preferred_element_type=jnp.float32)
                + bbuf[t]
            )
            pltpu.make_async_copy(obuf.at[oslot], o_hbm.at[t], so.at[oslot]).start()
        else:
            H = x_hbm.shape[1] // 2
            for h in range(2):
                rows = pl.ds(h * H, H)
                obuf[oslot, rows] = (
                    jnp.dot(xbuf[slot, rows], wbuf[slot],
                            preferred_element_type=jnp.float32)
                    + bbuf[t]
                )
                pltpu.make_async_copy(
                    obuf.at[oslot, rows], o_hbm.at[t, rows], so.at[oslot]
                ).start()
        if t + 3 < T:
            for c in in_copies(t + 3, slot):
                c.start()
    for t in (T - 3, T - 2):
        pltpu.make_async_copy(obuf.at[t % 3], o_hbm.at[t], so.at[t % 3]).wait()
    H = x_hbm.shape[1] // 2
    for h in range(2):
        rows = pl.ds(h * H, H)
        pltpu.make_async_copy(
            obuf.at[(T - 1) % 3, rows], o_hbm.at[T - 1, rows], so.at[(T - 1) % 3]
        ).wait()


def kernel(inputs, w, b):
    T, B, D_IN = inputs.shape
    D_OUT = w.shape[2]
    b3 = b.reshape(T, 1, D_OUT)
    return pl.pallas_call(
        _mm_pipe,
        in_specs=[
            pl.BlockSpec(memory_space=pltpu.HBM),
            pl.BlockSpec(memory_space=pltpu.HBM),
            pl.BlockSpec(memory_space=pltpu.HBM),
        ],
        out_specs=pl.BlockSpec(memory_space=pltpu.HBM),
        out_shape=jax.ShapeDtypeStruct((T, B, D_OUT), jnp.float32),
        scratch_shapes=[
            pltpu.VMEM((3, B, D_IN), jnp.float32),
            pltpu.VMEM((3, D_IN, D_OUT), jnp.float32),
            pltpu.VMEM((3, B, D_OUT), jnp.float32),
            pltpu.VMEM((T, 1, D_OUT), jnp.float32),
            pltpu.SemaphoreType.DMA((3,)),
            pltpu.SemaphoreType.DMA((3,)),
            pltpu.SemaphoreType.DMA((3,)),
            pltpu.SemaphoreType.DMA,
        ],
    )(inputs, w, b3)
